# Initial kernel scaffold; baseline (speedup 1.0000x reference)
import functools

import jax
import jax.numpy as jnp
from jax import lax
from jax.experimental import pallas as pl
from jax.experimental.pallas import tpu as pltpu
from jax.experimental.pallas import tpu_sc as plsc

B, L, V, D, DO = 16384, 200, 165, 4, 10
NC, NS = 2, 16
NW = NC * NS
RPW = B // NW
BLK = 16


def _make_kernel():
    mesh = plsc.VectorSubcoreMesh(
        core_axis_name="c", subcore_axis_name="s", num_cores=NC,
        num_subcores=NS)

    @functools.partial(
        pl.kernel,
        out_type=jax.ShapeDtypeStruct((B, DO), jnp.float32),
        mesh=mesh,
        scratch_types=[
            pltpu.VMEM((RPW, L), jnp.int32),
            pltpu.VMEM((V, D), jnp.float32),
            pltpu.VMEM((D, DO), jnp.float32),
            pltpu.VMEM((DO,), jnp.float32),
            pltpu.VMEM((RPW, DO), jnp.float32),
        ],
    )
    def showdown_kernel(x_hbm, tbl_hbm, w_hbm, b_hbm, out_hbm,
                        x_v, tbl_v, w_v, b_v, out_v):
        wid = lax.axis_index("s") * NC + lax.axis_index("c")
        base = wid * RPW
        pltpu.sync_copy(x_hbm.at[pl.ds(base, RPW)], x_v)
        pltpu.sync_copy(tbl_hbm, tbl_v)
        pltpu.sync_copy(w_hbm, w_v)
        pltpu.sync_copy(b_hbm, b_v)

        lane = lax.iota(jnp.int32, 16)
        wj = [[w_v[d, j] for j in range(DO)] for d in range(D)]
        bj = [b_v[j] for j in range(DO)]
        jvecs = [jnp.full((16,), j, jnp.int32) for j in range(DO)]

        def block_body(blk, _):
            rows = lane + blk * BLK
            zero = jnp.zeros((16,), jnp.float32)

            def l_body(l, accs):
                lsplat = jnp.full((16,), l, jnp.int32)
                idx = plsc.load_gather(x_v, [rows, lsplat])
                new = []
                for d in range(D):
                    dsplat = jnp.full((16,), d, jnp.int32)
                    g = plsc.load_gather(tbl_v, [idx, dsplat])
                    new.append(accs[d] + g)
                return tuple(new)

            accs = lax.fori_loop(0, L, l_body, (zero, zero, zero, zero))
            for j in range(DO):
                o = jnp.full((16,), bj[j], jnp.float32)
                for d in range(D):
                    o = o + accs[d] * wj[d][j]
                plsc.store_scatter(out_v, [rows, jvecs[j]], o)
            return 0

        lax.fori_loop(0, RPW // BLK, block_body, 0)
        pltpu.sync_copy(out_v, out_hbm.at[pl.ds(base, RPW)])

    return showdown_kernel


_kernel = _make_kernel()


def kernel(x, embed_table, W, b):
    return _kernel(x.astype(jnp.int32), embed_table, W, b)


# speedup vs baseline: 118.4544x; 118.4544x over previous
import functools

import jax
import jax.numpy as jnp
from jax import lax
from jax.experimental import pallas as pl
from jax.experimental.pallas import tpu as pltpu
from jax.experimental.pallas import tpu_sc as plsc

B, L, V, D, DO = 16384, 200, 165, 4, 10
NC, NS = 2, 16
NW = NC * NS
RPW = B // NW
BLK = 16


def _make_kernel():
    mesh = plsc.VectorSubcoreMesh(
        core_axis_name="c", subcore_axis_name="s", num_cores=NC,
        num_subcores=NS)

    @functools.partial(
        pl.kernel,
        out_type=jax.ShapeDtypeStruct((B, DO), jnp.float32),
        mesh=mesh,
        compiler_params=pltpu.CompilerParams(
            use_tc_tiling_on_sc=False, needs_layout_passes=False),
        scratch_types=[
            pltpu.VMEM((RPW, L), jnp.int32),
            pltpu.VMEM((V, D), jnp.float32),
            pltpu.VMEM((D, 16), jnp.float32),
            pltpu.VMEM((16,), jnp.float32),
            pltpu.VMEM((RPW, DO), jnp.float32),
        ],
    )
    def showdown_kernel(x_hbm, tbl_hbm, w_hbm, b_hbm, out_hbm,
                        x_v, tbl_v, w_v, b_v, out_v):
        wid = lax.axis_index("s") * NC + lax.axis_index("c")
        base = wid * RPW
        pltpu.sync_copy(x_hbm.at[pl.ds(base, RPW)], x_v)
        pltpu.sync_copy(tbl_hbm, tbl_v)
        pltpu.sync_copy(w_hbm, w_v)
        pltpu.sync_copy(b_hbm, b_v)

        lane = lax.iota(jnp.int32, 16)
        wrows = [w_v[d] for d in range(D)]
        brow = b_v[...]
        wj = [[wrows[d][j] for j in range(DO)] for d in range(D)]
        bj = [brow[j] for j in range(DO)]
        jvecs = [jnp.full((16,), j, jnp.int32) for j in range(DO)]

        def block_body(blk, _):
            rows = lane + blk * BLK
            zero = jnp.zeros((16,), jnp.float32)

            def l_body(l, accs):
                lsplat = jnp.full((16,), l, jnp.int32)
                idx = plsc.load_gather(x_v, [rows, lsplat])
                new = []
                for d in range(D):
                    dsplat = jnp.full((16,), d, jnp.int32)
                    g = plsc.load_gather(tbl_v, [idx, dsplat])
                    new.append(accs[d] + g)
                return tuple(new)

            accs = lax.fori_loop(0, L, l_body, (zero, zero, zero, zero))
            for j in range(DO):
                o = jnp.full((16,), bj[j], jnp.float32)
                for d in range(D):
                    o = o + accs[d] * wj[d][j]
                plsc.store_scatter(out_v, [rows, jvecs[j]], o)
            return 0

        lax.fori_loop(0, RPW // BLK, block_body, 0)
        pltpu.sync_copy(out_v, out_hbm.at[pl.ds(base, RPW)])

    return showdown_kernel


_kernel = _make_kernel()


def kernel(x, embed_table, W, b):
    w_pad = jnp.zeros((D, 16), jnp.float32).at[:, :DO].set(W)
    b_pad = jnp.zeros((16,), jnp.float32).at[:DO].set(b)
    return _kernel(x.astype(jnp.int32), embed_table, w_pad, b_pad)


# speedup vs baseline: 123.4426x; 1.0421x over previous
import functools

import jax
import jax.numpy as jnp
from jax import lax
from jax.experimental import pallas as pl
from jax.experimental.pallas import tpu as pltpu
from jax.experimental.pallas import tpu_sc as plsc

B, L, V, D, DO = 16384, 200, 165, 4, 10
NC, NS = 2, 16
NW = NC * NS
RPW = B // NW
BLK = 16
UNROLL = 8


def _make_kernel():
    mesh = plsc.VectorSubcoreMesh(
        core_axis_name="c", subcore_axis_name="s", num_cores=NC,
        num_subcores=NS)

    @functools.partial(
        pl.kernel,
        out_type=jax.ShapeDtypeStruct((B, DO), jnp.float32),
        mesh=mesh,
        compiler_params=pltpu.CompilerParams(
            use_tc_tiling_on_sc=False, needs_layout_passes=False),
        scratch_types=[
            pltpu.VMEM((RPW, L), jnp.int32),
            pltpu.VMEM((V, D), jnp.float32),
            pltpu.VMEM((D, 16), jnp.float32),
            pltpu.VMEM((16,), jnp.float32),
            pltpu.VMEM((RPW, DO), jnp.float32),
        ],
    )
    def showdown_kernel(x_hbm, tbl_hbm, w_hbm, b_hbm, out_hbm,
                        x_v, tbl_v, w_v, b_v, out_v):
        wid = lax.axis_index("s") * NC + lax.axis_index("c")
        base = wid * RPW
        pltpu.sync_copy(x_hbm.at[pl.ds(base, RPW)], x_v)
        pltpu.sync_copy(tbl_hbm, tbl_v)
        pltpu.sync_copy(w_hbm, w_v)
        pltpu.sync_copy(b_hbm, b_v)

        lane = lax.iota(jnp.int32, 16)
        wrows = [w_v[d] for d in range(D)]
        brow = b_v[...]
        wj = [[wrows[d][j] for j in range(DO)] for d in range(D)]
        bj = [brow[j] for j in range(DO)]
        jvecs = [jnp.full((16,), j, jnp.int32) for j in range(DO)]

        def block_body(blk, _):
            rows = lane + blk * BLK
            zero = jnp.zeros((16,), jnp.float32)

            dsplats = [jnp.full((16,), d, jnp.int32) for d in range(D)]

            def l_body(i, accs):
                accs = list(accs)
                lbase = i * UNROLL
                for k in range(UNROLL):
                    lsplat = jnp.full((16,), lbase + k, jnp.int32)
                    idx = plsc.load_gather(x_v, [rows, lsplat])
                    for d in range(D):
                        g = plsc.load_gather(tbl_v, [idx, dsplats[d]])
                        accs[d] = accs[d] + g
                return tuple(accs)

            accs = lax.fori_loop(0, L // UNROLL, l_body,
                                 (zero, zero, zero, zero))
            for j in range(DO):
                o = jnp.full((16,), bj[j], jnp.float32)
                for d in range(D):
                    o = o + accs[d] * wj[d][j]
                plsc.store_scatter(out_v, [rows, jvecs[j]], o)
            return 0

        lax.fori_loop(0, RPW // BLK, block_body, 0)
        pltpu.sync_copy(out_v, out_hbm.at[pl.ds(base, RPW)])

    return showdown_kernel


_kernel = _make_kernel()


def kernel(x, embed_table, W, b):
    w_pad = jnp.zeros((D, 16), jnp.float32).at[:, :DO].set(W)
    b_pad = jnp.zeros((16,), jnp.float32).at[:DO].set(b)
    return _kernel(x.astype(jnp.int32), embed_table, w_pad, b_pad)


# speedup vs baseline: 131.3075x; 1.0637x over previous
import functools

import jax
import jax.numpy as jnp
from jax import lax
from jax.experimental import pallas as pl
from jax.experimental.pallas import tpu as pltpu
from jax.experimental.pallas import tpu_sc as plsc

B, L, V, D, DO = 16384, 200, 165, 4, 10
NC, NS = 2, 16
NW = NC * NS
RPW = B // NW
BLK = 16
UNROLL = 8
LP = L + 1


def _make_kernel():
    mesh = plsc.VectorSubcoreMesh(
        core_axis_name="c", subcore_axis_name="s", num_cores=NC,
        num_subcores=NS)

    @functools.partial(
        pl.kernel,
        out_type=jax.ShapeDtypeStruct((B, DO), jnp.float32),
        mesh=mesh,
        compiler_params=pltpu.CompilerParams(
            use_tc_tiling_on_sc=False, needs_layout_passes=False),
        scratch_types=[
            pltpu.VMEM((RPW, LP), jnp.int32),
            pltpu.VMEM((V * D * 16,), jnp.float32),
            pltpu.VMEM((D, 16), jnp.float32),
            pltpu.VMEM((16,), jnp.float32),
            pltpu.VMEM((RPW, DO), jnp.float32),
        ],
    )
    def showdown_kernel(x_hbm, tbl_hbm, w_hbm, b_hbm, out_hbm,
                        x_v, tbl_v, w_v, b_v, out_v):
        wid = lax.axis_index("s") * NC + lax.axis_index("c")
        base = wid * RPW
        pltpu.sync_copy(x_hbm.at[pl.ds(base, RPW)], x_v.at[:, pl.ds(0, L)])
        pltpu.sync_copy(tbl_hbm, tbl_v)
        pltpu.sync_copy(w_hbm, w_v)
        pltpu.sync_copy(b_hbm, b_v)

        lane = lax.iota(jnp.int32, 16)
        wrows = [w_v[d] for d in range(D)]
        brow = b_v[...]
        wj = [[wrows[d][j] for j in range(DO)] for d in range(D)]
        bj = [brow[j] for j in range(DO)]
        jvecs = [jnp.full((16,), j, jnp.int32) for j in range(DO)]
        doffs = [lane + d * 16 for d in range(D)]

        def block_body(blk, _):
            rows = lane + blk * BLK
            zero = jnp.zeros((16,), jnp.float32)

            def l_body(i, accs):
                accs = list(accs)
                lbase = i * UNROLL
                for k in range(UNROLL):
                    lsplat = jnp.full((16,), lbase + k, jnp.int32)
                    idx = plsc.load_gather(x_v, [rows, lsplat])
                    idx64 = idx * (D * 16)
                    for d in range(D):
                        g = plsc.load_gather(tbl_v, [idx64 + doffs[d]])
                        accs[d] = accs[d] + g
                return tuple(accs)

            accs = lax.fori_loop(0, L // UNROLL, l_body,
                                 (zero, zero, zero, zero))
            for j in range(DO):
                o = jnp.full((16,), bj[j], jnp.float32)
                for d in range(D):
                    o = o + accs[d] * wj[d][j]
                plsc.store_scatter(out_v, [rows, jvecs[j]], o)
            return 0

        lax.fori_loop(0, RPW // BLK, block_body, 0)
        pltpu.sync_copy(out_v, out_hbm.at[pl.ds(base, RPW)])

    return showdown_kernel


_kernel = _make_kernel()


def kernel(x, embed_table, W, b):
    tbl_rep = jnp.broadcast_to(
        embed_table.reshape(V * D, 1), (V * D, 16)).reshape(-1)
    w_pad = jnp.zeros((D, 16), jnp.float32).at[:, :DO].set(W)
    b_pad = jnp.zeros((16,), jnp.float32).at[:DO].set(b)
    return _kernel(x.astype(jnp.int32), tbl_rep, w_pad, b_pad)
